# Initial kernel scaffold; baseline (speedup 1.0000x reference)
import jax
import jax.numpy as jnp
from jax.experimental import pallas as pl


def kernel(points_data, W1, b1, g1, be1, W2, b2, g2, be2, W3, b3, g3, be3):
    raise NotImplementedError("write your pallas kernel here")


# speedup vs baseline: 1.7457x; 1.7457x over previous
import jax
import jax.numpy as jnp
from jax import lax
from jax.experimental import pallas as pl
from jax.experimental.pallas import tpu as pltpu

B = 8
N = 8192
N_GROUPS = 512
GROUP_SIZE = 32
EMBED_DIM = 384


def _fps_body(xyz_ref, f0_ref, cx_ref, cy_ref, cz_ref):
    x = xyz_ref[0]
    y = xyz_ref[1]
    z = xyz_ref[2]
    col = lax.broadcasted_iota(jnp.int32, (B, N), 1)
    colM = lax.broadcasted_iota(jnp.int32, (B, N_GROUPS), 1)

    def body(i, carry):
        dist, far, ax, ay, az = carry
        onehot = (col == far).astype(jnp.float32)
        cx = jnp.sum(x * onehot, axis=1, keepdims=True)
        cy = jnp.sum(y * onehot, axis=1, keepdims=True)
        cz = jnp.sum(z * onehot, axis=1, keepdims=True)
        hit = colM == i
        ax = jnp.where(hit, cx, ax)
        ay = jnp.where(hit, cy, ay)
        az = jnp.where(hit, cz, az)
        dx = x - cx
        dy = y - cy
        dz = z - cz
        d = dx * dx + dy * dy
        d = d + dz * dz
        dist = jnp.where(d < dist, d, dist)
        m = jnp.max(dist, axis=1, keepdims=True)
        sel = jnp.where(dist == m, col, jnp.int32(N))
        far = jnp.min(sel, axis=1, keepdims=True)
        return dist, far, ax, ay, az

    dist0 = jnp.full((B, N), 1e10, dtype=jnp.float32)
    far0 = f0_ref[...]
    zM = jnp.zeros((B, N_GROUPS), dtype=jnp.float32)
    _, _, ax, ay, az = lax.fori_loop(0, N_GROUPS, body,
                                     (dist0, far0, zM, zM, zM))
    cx_ref[...] = ax
    cy_ref[...] = ay
    cz_ref[...] = az


def _fps_centroids(xyz_t, f0):
    cx, cy, cz = pl.pallas_call(
        _fps_body,
        out_shape=[jax.ShapeDtypeStruct((B, N_GROUPS), jnp.float32)] * 3,
    )(xyz_t, f0)
    return jnp.stack([cx, cy, cz], axis=-1)


def kernel(points_data, W1, b1, g1, be1, W2, b2, g2, be2, W3, b3, g3, be3):
    xyz = points_data
    xyz_t = jnp.transpose(xyz, (2, 0, 1))
    f0 = jax.random.randint(jax.random.key(42), (B,), 0, N,
                            dtype=jnp.int32).reshape(B, 1)
    centroids_xyz = _fps_centroids(xyz_t, f0)

    d2 = (jnp.sum(centroids_xyz ** 2, axis=-1)[:, :, None]
          + jnp.sum(xyz ** 2, axis=-1)[:, None, :]
          - 2.0 * jnp.einsum('bmc,bnc->bmn', centroids_xyz, xyz))
    _, idx = lax.top_k(-d2, GROUP_SIZE)
    grouped = jnp.take_along_axis(
        xyz, idx.reshape(B, N_GROUPS * GROUP_SIZE)[:, :, None], axis=1)
    grouped = grouped.reshape(B, N_GROUPS, GROUP_SIZE, 3)
    gn = grouped - centroids_xyz[:, :, None, :]

    def bn(yv, g, be, eps=1e-5):
        mean = jnp.mean(yv, axis=(0, 2), keepdims=True)
        var = jnp.var(yv, axis=(0, 2), keepdims=True)
        return g[None, :, None] * (yv - mean) / jnp.sqrt(var + eps) + be[None, :, None]

    xg = jnp.transpose(gn, (0, 1, 3, 2)).reshape(-1, 3, GROUP_SIZE)
    yv = jnp.einsum('oc,bck->bok', W1, xg) + b1[None, :, None]
    yv = jax.nn.relu(bn(yv, g1, be1))
    yv = jnp.einsum('oc,bck->bok', W2, yv) + b2[None, :, None]
    yv = jax.nn.relu(bn(yv, g2, be2))
    yv = jnp.einsum('oc,bck->bok', W3, yv) + b3[None, :, None]
    yv = bn(yv, g3, be3)
    tokens = jnp.max(yv, axis=2).reshape(B, N_GROUPS, EMBED_DIM)
    return (tokens, centroids_xyz)
